# baseline (speedup 1.0000x reference)
import functools

import jax
import jax.numpy as jnp
from jax import lax
from jax.experimental import pallas as pl
from jax.experimental.pallas import tpu as pltpu
from jax.experimental.pallas import tpu_sc as plsc

N = 10000
NUM_POS = 640000
NUM_NEG = 640000
NE = NUM_POS + NUM_NEG

NC, NS, L = 2, 16, 16
NW = NC * NS
CH = NUM_POS // NS


def _sc_gather_body(flat_ref, pos_ref, neg_ref, out_ref, rbuf, cbuf, vbuf, sem):
    wid = lax.axis_index("s") * NC + lax.axis_index("c")
    half = wid // NS
    slot = wid % NS
    base = slot * CH

    @pl.when(half == 0)
    def _():
        pltpu.sync_copy(pos_ref.at[pl.ds(base, CH)], rbuf)
        pltpu.sync_copy(pos_ref.at[pl.ds(NUM_POS + base, CH)], cbuf)

    @pl.when(half == 1)
    def _():
        pltpu.sync_copy(neg_ref.at[pl.ds(base, CH)], rbuf)
        pltpu.sync_copy(neg_ref.at[pl.ds(NUM_NEG + base, CH)], cbuf)

    def flat_idx(i, _):
        off = i * L
        r = rbuf[pl.ds(off, L)]
        c = cbuf[pl.ds(off, L)]
        rbuf[pl.ds(off, L)] = r * N + c
        return _

    lax.fori_loop(0, CH // L, flat_idx, None)

    pltpu.async_copy(flat_ref.at[rbuf], vbuf, sem).wait()

    obase = half * NUM_POS + base
    pltpu.sync_copy(vbuf, out_ref.at[pl.ds(obase, CH)])


@functools.partial(
    pl.kernel,
    out_type=jax.ShapeDtypeStruct((NE,), jnp.float32),
    mesh=plsc.VectorSubcoreMesh(core_axis_name="c", subcore_axis_name="s"),
    scratch_types=[
        pltpu.VMEM((CH,), jnp.int32),
        pltpu.VMEM((CH,), jnp.int32),
        pltpu.VMEM((CH,), jnp.float32),
        pltpu.SemaphoreType.DMA,
    ],
)
def _sc_gather(flat_ref, pos_ref, neg_ref, out_ref, rbuf, cbuf, vbuf, sem):
    _sc_gather_body(flat_ref, pos_ref, neg_ref, out_ref, rbuf, cbuf, vbuf, sem)


ROWS = NE // 128
GRID = 10
RB = ROWS // GRID


def _bce_reduce_body(x_ref, o_ref):
    pid = pl.program_id(0)
    sign = jnp.where(pid < GRID // 2, -1.0, 1.0)
    z = x_ref[...] * sign
    loss = jnp.maximum(z, 0.0) + jnp.log1p(jnp.exp(-jnp.abs(z)))
    part = jnp.sum(loss)

    @pl.when(pid == 0)
    def _():
        o_ref[0, 0] = 0.0

    o_ref[0, 0] += part


def _bce_mean(logits):
    x2d = logits.reshape(ROWS, 128)
    out = pl.pallas_call(
        _bce_reduce_body,
        grid=(GRID,),
        in_specs=[pl.BlockSpec((RB, 128), lambda i: (i, 0))],
        out_specs=pl.BlockSpec((1, 1), lambda i: (0, 0), memory_space=pltpu.SMEM),
        out_shape=jax.ShapeDtypeStruct((1, 1), jnp.float32),
    )(x2d)
    return out[0, 0] / NE


def kernel(input, pos_edge_index, neg_edge_index):
    flat = input.reshape(-1)
    pos = pos_edge_index.astype(jnp.int32).reshape(-1)
    neg = neg_edge_index.astype(jnp.int32).reshape(-1)
    logits = _sc_gather(flat, pos, neg)
    return _bce_mean(logits)


# speedup vs baseline: 1.0322x; 1.0322x over previous
import functools

import jax
import jax.numpy as jnp
from jax import lax
from jax.experimental import pallas as pl
from jax.experimental.pallas import tpu as pltpu
from jax.experimental.pallas import tpu_sc as plsc

N = 10000
NUM_POS = 640000
NUM_NEG = 640000
NE = NUM_POS + NUM_NEG

NC, NS, L = 2, 16, 16
NW = NC * NS
CH = NUM_POS // NS
NWIN = 10
WIN = CH // NWIN


def _sc_gather_body(flat_ref, pos_ref, neg_ref, out_ref, rbuf, cbuf, vbuf, sem, isem):
    wid = lax.axis_index("s") * NC + lax.axis_index("c")
    half = wid // NS
    slot = wid % NS
    base = slot * CH

    @pl.when(half == 0)
    def _():
        c1 = pltpu.async_copy(pos_ref.at[pl.ds(base, CH)], rbuf, isem)
        c2 = pltpu.async_copy(pos_ref.at[pl.ds(NUM_POS + base, CH)], cbuf, isem)
        c1.wait()
        c2.wait()

    @pl.when(half == 1)
    def _():
        c1 = pltpu.async_copy(neg_ref.at[pl.ds(base, CH)], rbuf, isem)
        c2 = pltpu.async_copy(neg_ref.at[pl.ds(NUM_NEG + base, CH)], cbuf, isem)
        c1.wait()
        c2.wait()

    obase = half * NUM_POS + base
    gathers = []
    for j in range(NWIN):
        o = j * WIN

        @plsc.parallel_loop(o, o + WIN, step=L, unroll=10)
        def _flat(i):
            rbuf[pl.ds(i, L)] = rbuf[pl.ds(i, L)] * N + cbuf[pl.ds(i, L)]

        gathers.append(
            pltpu.async_copy(
                flat_ref.at[rbuf.at[pl.ds(o, WIN)]], vbuf.at[pl.ds(o, WIN)], sem))

    writes = []
    for j, g in enumerate(gathers):
        o = j * WIN
        g.wait()
        writes.append(
            pltpu.async_copy(vbuf.at[pl.ds(o, WIN)], out_ref.at[pl.ds(obase + o, WIN)], isem))
    for w in writes:
        w.wait()


@functools.partial(
    pl.kernel,
    out_type=jax.ShapeDtypeStruct((NE,), jnp.float32),
    mesh=plsc.VectorSubcoreMesh(core_axis_name="c", subcore_axis_name="s"),
    scratch_types=[
        pltpu.VMEM((CH,), jnp.int32),
        pltpu.VMEM((CH,), jnp.int32),
        pltpu.VMEM((CH,), jnp.float32),
        pltpu.SemaphoreType.DMA,
        pltpu.SemaphoreType.DMA,
    ],
)
def _sc_gather(flat_ref, pos_ref, neg_ref, out_ref, rbuf, cbuf, vbuf, sem, isem):
    _sc_gather_body(flat_ref, pos_ref, neg_ref, out_ref, rbuf, cbuf, vbuf, sem, isem)


ROWS = NE // 128
GRID = 10
RB = ROWS // GRID


def _bce_reduce_body(x_ref, o_ref):
    pid = pl.program_id(0)
    sign = jnp.where(pid < GRID // 2, -1.0, 1.0)
    z = x_ref[...] * sign
    loss = jnp.maximum(z, 0.0) + jnp.log1p(jnp.exp(-jnp.abs(z)))
    part = jnp.sum(loss)

    @pl.when(pid == 0)
    def _():
        o_ref[0, 0] = 0.0

    o_ref[0, 0] += part


def _bce_mean(logits):
    x2d = logits.reshape(ROWS, 128)
    out = pl.pallas_call(
        _bce_reduce_body,
        grid=(GRID,),
        in_specs=[pl.BlockSpec((RB, 128), lambda i: (i, 0))],
        out_specs=pl.BlockSpec((1, 1), lambda i: (0, 0), memory_space=pltpu.SMEM),
        out_shape=jax.ShapeDtypeStruct((1, 1), jnp.float32),
    )(x2d)
    return out[0, 0] / NE


def kernel(input, pos_edge_index, neg_edge_index):
    flat = input.reshape(-1)
    pos = pos_edge_index.astype(jnp.int32).reshape(-1)
    neg = neg_edge_index.astype(jnp.int32).reshape(-1)
    logits = _sc_gather(flat, pos, neg)
    return _bce_mean(logits)


# speedup vs baseline: 1.0366x; 1.0043x over previous
import functools

import jax
import jax.numpy as jnp
from jax import lax
from jax.experimental import pallas as pl
from jax.experimental.pallas import tpu as pltpu
from jax.experimental.pallas import tpu_sc as plsc

N = 10000
NUM_POS = 640000
NUM_NEG = 640000
NE = NUM_POS + NUM_NEG

NC, NS, L = 2, 16, 16
NW = NC * NS
CH = NUM_POS // NS
NWIN = 10
WIN = CH // NWIN


def _softplus16(z):
    u = jnp.exp(-jnp.abs(z))
    s = u / (u + 2.0)
    t = s * s
    log1p = 2.0 * s * (1.0 + t * (1.0 / 3.0 + t * (0.2 + t * (1.0 / 7.0))))
    return jnp.maximum(z, 0.0) + log1p


def _sc_body(flat_ref, pos_ref, neg_ref, out_ref, rbuf, cbuf, vbuf, abuf, sem, isem):
    wid = lax.axis_index("s") * NC + lax.axis_index("c")
    half = wid // NS
    slot = wid % NS
    base = slot * CH

    @pl.when(half == 0)
    def _():
        c1 = pltpu.async_copy(pos_ref.at[pl.ds(base, CH)], rbuf, isem)
        c2 = pltpu.async_copy(pos_ref.at[pl.ds(NUM_POS + base, CH)], cbuf, isem)
        c1.wait()
        c2.wait()

    @pl.when(half == 1)
    def _():
        c1 = pltpu.async_copy(neg_ref.at[pl.ds(base, CH)], rbuf, isem)
        c2 = pltpu.async_copy(neg_ref.at[pl.ds(NUM_NEG + base, CH)], cbuf, isem)
        c1.wait()
        c2.wait()

    sign = jnp.where(half == 0, -1.0, 1.0)

    gathers = []
    for j in range(NWIN):
        o = j * WIN

        @plsc.parallel_loop(o, o + WIN, step=L, unroll=10)
        def _flat(i):
            rbuf[pl.ds(i, L)] = rbuf[pl.ds(i, L)] * N + cbuf[pl.ds(i, L)]

        gathers.append(
            pltpu.async_copy(
                flat_ref.at[rbuf.at[pl.ds(o, WIN)]], vbuf.at[pl.ds(o, WIN)], sem))

    acc = jnp.zeros((L,), jnp.float32)
    for j, g in enumerate(gathers):
        o = j * WIN
        g.wait()

        @plsc.parallel_loop(o, o + WIN, step=L, unroll=5, carry=acc)
        def _acc(i, a):
            z = vbuf[pl.ds(i, L)] * sign
            return a + _softplus16(z)

        acc = _acc

    abuf[...] = acc
    pltpu.sync_copy(abuf, out_ref.at[pl.ds(wid * L, L)])


@functools.partial(
    pl.kernel,
    out_type=jax.ShapeDtypeStruct((NW * L,), jnp.float32),
    mesh=plsc.VectorSubcoreMesh(core_axis_name="c", subcore_axis_name="s"),
    scratch_types=[
        pltpu.VMEM((CH,), jnp.int32),
        pltpu.VMEM((CH,), jnp.int32),
        pltpu.VMEM((CH,), jnp.float32),
        pltpu.VMEM((L,), jnp.float32),
        pltpu.SemaphoreType.DMA,
        pltpu.SemaphoreType.DMA,
    ],
)
def _sc_loss(flat_ref, pos_ref, neg_ref, out_ref, rbuf, cbuf, vbuf, abuf, sem, isem):
    _sc_body(flat_ref, pos_ref, neg_ref, out_ref, rbuf, cbuf, vbuf, abuf, sem, isem)


def kernel(input, pos_edge_index, neg_edge_index):
    flat = input.reshape(-1)
    pos = pos_edge_index.astype(jnp.int32).reshape(-1)
    neg = neg_edge_index.astype(jnp.int32).reshape(-1)
    partials = _sc_loss(flat, pos, neg)
    return jnp.sum(partials) / NE
